# baseline (speedup 1.0000x reference)
import functools

import jax
import jax.numpy as jnp
from jax.experimental import pallas as pl

N = 10000
E = 160000
T = 640000
NG = 64
HC = 128
IE = 64
OE = 256

EB = 2000
TB = 4000
NB = 2000


def _swish(x):
    return x * jax.nn.sigmoid(x)


def _dot(a, b):
    return jnp.dot(a, b, preferred_element_type=jnp.float32)


def _full(shape):
    return pl.BlockSpec(shape, lambda e: tuple(0 for _ in shape))


def _rows(nrows, ncols):
    return pl.BlockSpec((nrows, ncols), lambda e: (e, 0))


def _node_body(x_ref, wa_ref, wb_ref, xa_ref, xb_ref):
    x = x_ref[...]
    xa_ref[...] = _dot(x, wa_ref[...])
    xb_ref[...] = _dot(x, wb_ref[...])


def _node_proj(x, wa, wb):
    return pl.pallas_call(
        _node_body,
        grid=(N // NB,),
        in_specs=[_rows(NB, HC), _full((HC, HC)), _full((HC, HC))],
        out_specs=[_rows(NB, HC), _rows(NB, HC)],
        out_shape=[jax.ShapeDtypeStruct((N, HC), jnp.float32)] * 2,
    )(x, wa, wb)


def _init_edge_body(gxa_ref, gxb_ref, rbfp_ref, w0_ref, b0_ref, wc_ref,
                    bc_ref, wr1_ref, e1_ref, e2_ref, rbf0_ref):
    rbfp = rbfp_ref[...]
    rbf0 = _swish(_dot(rbfp, w0_ref[...]) + b0_ref[...])
    e1 = _swish(gxa_ref[...] + gxb_ref[...] + _dot(rbf0, wc_ref[...])
                + bc_ref[...])
    e1_ref[...] = e1
    e2_ref[...] = _dot(rbfp, wr1_ref[...]) * e1
    rbf0_ref[...] = rbf0


def _init_edge(gxa, gxb, rbfp, w0, b0, wc, bc, wr1):
    return pl.pallas_call(
        _init_edge_body,
        grid=(E // EB,),
        in_specs=[_rows(EB, HC), _rows(EB, HC), _rows(EB, 8),
                  _full((8, HC)), _full((1, HC)), _full((HC, HC)),
                  _full((1, HC)), _full((8, HC))],
        out_specs=[_rows(EB, HC)] * 3,
        out_shape=[jax.ShapeDtypeStruct((E, HC), jnp.float32)] * 3,
    )(gxa, gxb, rbfp, w0, b0, wc, bc, wr1)


def _edge_pre_body(x1_ref, rbfp_ref, wji_ref, bji_ref, wkj_ref, bkj_ref,
                   m_ref, wdown_ref, xji_ref, xdown_ref):
    x1 = x1_ref[...]
    xji_ref[...] = _swish(_dot(x1, wji_ref[...]) + bji_ref[...])
    x_kj = _swish(_dot(x1, wkj_ref[...]) + bkj_ref[...])
    x_kj = x_kj * _dot(rbfp_ref[...], m_ref[...])
    xdown_ref[...] = _swish(_dot(x_kj, wdown_ref[...]))


def _edge_pre(x1, rbfp, wji, bji, wkj, bkj, m, wdown):
    return pl.pallas_call(
        _edge_pre_body,
        grid=(E // EB,),
        in_specs=[_rows(EB, HC), _rows(EB, 8), _full((HC, HC)),
                  _full((1, HC)), _full((HC, HC)), _full((1, HC)),
                  _full((8, HC)), _full((HC, IE))],
        out_specs=[_rows(EB, HC), _rows(EB, IE)],
        out_shape=[jax.ShapeDtypeStruct((E, HC), jnp.float32),
                   jax.ShapeDtypeStruct((E, IE), jnp.float32)],
    )(x1, rbfp, wji, bji, wkj, bkj, m, wdown)


def _edge_post_body(agg_ref, xji_ref, x1_ref, rbfp_ref, wup_ref, *rest):
    (wb1, bb1, wb2, bb2, wl, bl, wa1, ba1, wa2, ba2, wa3, ba3, wa4, ba4,
     wrbf, e1_ref, e2_ref) = rest
    xkj = _swish(_dot(agg_ref[...], wup_ref[...]))
    e1 = xji_ref[...] + xkj
    e1 = e1 + _swish(_dot(_swish(_dot(e1, wb1[...]) + bb1[...]), wb2[...])
                     + bb2[...])
    e1 = _swish(_dot(e1, wl[...]) + bl[...]) + x1_ref[...]
    e1 = e1 + _swish(_dot(_swish(_dot(e1, wa1[...]) + ba1[...]), wa2[...])
                     + ba2[...])
    e1 = e1 + _swish(_dot(_swish(_dot(e1, wa3[...]) + ba3[...]), wa4[...])
                     + ba4[...])
    e1_ref[...] = e1
    e2_ref[...] = _dot(rbfp_ref[...], wrbf[...]) * e1


def _edge_post(agg, xji, x1, rbfp, wup, wlist):
    specs = ([_rows(EB, IE), _rows(EB, HC), _rows(EB, HC), _rows(EB, 8),
              _full((IE, HC))]
             + [_full((HC, HC)), _full((1, HC))] * 7
             + [_full((8, HC))])
    return pl.pallas_call(
        _edge_post_body,
        grid=(E // EB,),
        in_specs=specs,
        out_specs=[_rows(EB, HC)] * 2,
        out_shape=[jax.ShapeDtypeStruct((E, HC), jnp.float32)] * 2,
    )(agg, xji, x1, rbfp, wup, *wlist)


def _sproj_body(sbfp_ref, m_ref, out_ref):
    out_ref[...] = _dot(sbfp_ref[...], m_ref[...])


def _sbf_proj(sbfp, m):
    return pl.pallas_call(
        _sproj_body,
        grid=(T // TB,),
        in_specs=[_rows(TB, 48), _full((48, IE))],
        out_specs=_rows(TB, IE),
        out_shape=jax.ShapeDtypeStruct((T, IE), jnp.float32),
    )(sbfp, m)


def _tmul_body(g_ref, s_ref, out_ref):
    out_ref[...] = g_ref[...] * s_ref[...]


def _tmul(g, s):
    return pl.pallas_call(
        _tmul_body,
        grid=(T // TB,),
        in_specs=[_rows(TB, IE), _rows(TB, IE)],
        out_specs=_rows(TB, IE),
        out_shape=jax.ShapeDtypeStruct((T, IE), jnp.float32),
    )(g, s)


def _vmlp_body(vagg_ref, batch_ref, wup_ref, bup_ref, w1, b1, w2, b2, w3,
               b3, wo, u_ref):
    v = _dot(vagg_ref[...], wup_ref[...]) + bup_ref[...]
    v = _swish(_dot(v, w1[...]) + b1[...])
    v = _swish(_dot(v, w2[...]) + b2[...])
    v = _swish(_dot(v, w3[...]) + b3[...])
    v = _dot(v, wo[...])
    seg = batch_ref[...]
    onehot = (seg == jax.lax.broadcasted_iota(jnp.int32, (NB, NG), 1)
              ).astype(jnp.float32)
    part = jax.lax.dot_general(onehot, v, (((0,), (0,)), ((), ())),
                               preferred_element_type=jnp.float32)

    @pl.when(pl.program_id(0) == 0)
    def _():
        u_ref[...] = jnp.zeros_like(u_ref)

    u_ref[...] += part


def _vmlp(vagg, batch2d, V):
    wlist = [V['lin_up']['w'], V['lin_up']['b'].reshape(1, OE)]
    for l in V['lins']:
        wlist += [l['w'], l['b'].reshape(1, OE)]
    wlist += [V['lin']['w']]
    specs = ([_rows(NB, HC), _rows(NB, 1), _full((HC, OE)), _full((1, OE))]
             + [_full((OE, OE)), _full((1, OE))] * 3
             + [_full((OE, 1))])
    return pl.pallas_call(
        _vmlp_body,
        grid=(N // NB,),
        in_specs=specs,
        out_specs=_full((NG, 1)),
        out_shape=jax.ShapeDtypeStruct((NG, 1), jnp.float32),
    )(vagg, batch2d, *wlist)


def kernel(z, rbf, sbf, i, j, idx_kj, idx_ji, batch, params):
    P = params
    rbfp = jnp.pad(rbf, ((0, 0), (0, 2)))
    sbfp = jnp.pad(sbf, ((0, 0), (0, 6)))
    batch2d = batch.astype(jnp.int32).reshape(N, 1)

    x = P['emb_table'][z]

    Wfull = P['init']['lin']['w']
    wa, wb, wc = Wfull[:HC], Wfull[HC:2 * HC], Wfull[2 * HC:]
    xa, xb = _node_proj(x, wa, wb)
    gxa = jnp.take(xa, i, axis=0)
    gxb = jnp.take(xb, j, axis=0)
    w0 = jnp.pad(P['init']['lin_rbf_0']['w'], ((0, 2), (0, 0)))
    b0 = P['init']['lin_rbf_0']['b'].reshape(1, HC)
    bc = P['init']['lin']['b'].reshape(1, HC)
    wr1 = jnp.pad(P['init']['lin_rbf_1']['w'], ((0, 2), (0, 0)))
    e1, e2, rbf0 = _init_edge(gxa, gxb, rbfp, w0, b0, Wfull[2 * HC:], bc,
                              wr1)

    V0 = P['update_v'][0]
    vagg = jax.ops.segment_sum(e2, i, num_segments=N)
    u = _vmlp(vagg, batch2d, V0)

    for li in range(4):
        L = P['update_e'][li]
        m_rbf = jnp.pad(_dot(L['lin_rbf1']['w'], L['lin_rbf2']['w']),
                        ((0, 2), (0, 0)))
        m_sbf = jnp.pad(_dot(L['lin_sbf1']['w'], L['lin_sbf2']['w']),
                        ((0, 6), (0, 0)))
        w_rbf = jnp.pad(L['lin_rbf']['w'], ((0, 2), (0, 0)))

        xji, xdown = _edge_pre(
            e1, rbfp, L['lin_ji']['w'], L['lin_ji']['b'].reshape(1, HC),
            L['lin_kj']['w'], L['lin_kj']['b'].reshape(1, HC), m_rbf,
            L['lin_down']['w'])

        s = _sbf_proj(sbfp, m_sbf)
        g = jnp.take(xdown, idx_kj, axis=0)
        prod = _tmul(g, s)
        agg = jax.ops.segment_sum(prod, idx_ji, num_segments=E)

        wlist = []
        for rl in L['before']:
            wlist += [rl['lin1']['w'], rl['lin1']['b'].reshape(1, HC),
                      rl['lin2']['w'], rl['lin2']['b'].reshape(1, HC)]
        wlist += [L['lin']['w'], L['lin']['b'].reshape(1, HC)]
        for rl in L['after']:
            wlist += [rl['lin1']['w'], rl['lin1']['b'].reshape(1, HC),
                      rl['lin2']['w'], rl['lin2']['b'].reshape(1, HC)]
        wlist += [w_rbf]
        e1, e2 = _edge_post(agg, xji, e1, rbfp, L['lin_up']['w'], wlist)

        vagg = jax.ops.segment_sum(e2, i, num_segments=N)
        u = u + _vmlp(vagg, batch2d, P['update_v'][li + 1])

    return u


# speedup vs baseline: 4.1164x; 4.1164x over previous
import functools

import jax
import jax.numpy as jnp
from jax import lax
from jax.experimental import pallas as pl
from jax.experimental.pallas import tpu as pltpu
from jax.experimental.pallas import tpu_sc as plsc

N = 10000
E = 160000
T = 640000
NG = 64
HC = 128
IE = 64
OE = 256

EB = 2000
TB = 4000
NB = 2000


def _swish(x):
    return x * jax.nn.sigmoid(x)


def _dot(a, b):
    return jnp.dot(a, b, preferred_element_type=jnp.float32)


def _full(shape):
    return pl.BlockSpec(shape, lambda e: tuple(0 for _ in shape))


def _rows(nrows, ncols):
    return pl.BlockSpec((nrows, ncols), lambda e: (e, 0))


def _node_body(x_ref, wa_ref, wb_ref, xa_ref, xb_ref):
    x = x_ref[...]
    xa_ref[...] = _dot(x, wa_ref[...])
    xb_ref[...] = _dot(x, wb_ref[...])


def _node_proj(x, wa, wb):
    return pl.pallas_call(
        _node_body,
        grid=(N // NB,),
        in_specs=[_rows(NB, HC), _full((HC, HC)), _full((HC, HC))],
        out_specs=[_rows(NB, HC), _rows(NB, HC)],
        out_shape=[jax.ShapeDtypeStruct((N, HC), jnp.float32)] * 2,
    )(x, wa, wb)


def _init_edge_body(gxa_ref, gxb_ref, rbfp_ref, w0_ref, b0_ref, wc_ref,
                    bc_ref, wr1_ref, e1_ref, e2_ref, rbf0_ref):
    rbfp = rbfp_ref[...]
    rbf0 = _swish(_dot(rbfp, w0_ref[...]) + b0_ref[...])
    e1 = _swish(gxa_ref[...] + gxb_ref[...] + _dot(rbf0, wc_ref[...])
                + bc_ref[...])
    e1_ref[...] = e1
    e2_ref[...] = _dot(rbfp, wr1_ref[...]) * e1
    rbf0_ref[...] = rbf0


def _init_edge(gxa, gxb, rbfp, w0, b0, wc, bc, wr1):
    return pl.pallas_call(
        _init_edge_body,
        grid=(E // EB,),
        in_specs=[_rows(EB, HC), _rows(EB, HC), _rows(EB, 8),
                  _full((8, HC)), _full((1, HC)), _full((HC, HC)),
                  _full((1, HC)), _full((8, HC))],
        out_specs=[_rows(EB, HC)] * 3,
        out_shape=[jax.ShapeDtypeStruct((E, HC), jnp.float32)] * 3,
    )(gxa, gxb, rbfp, w0, b0, wc, bc, wr1)


def _edge_pre_body(x1_ref, rbfp_ref, wji_ref, bji_ref, wkj_ref, bkj_ref,
                   m_ref, wdown_ref, xji_ref, xdown_ref):
    x1 = x1_ref[...]
    xji_ref[...] = _swish(_dot(x1, wji_ref[...]) + bji_ref[...])
    x_kj = _swish(_dot(x1, wkj_ref[...]) + bkj_ref[...])
    x_kj = x_kj * _dot(rbfp_ref[...], m_ref[...])
    xdown_ref[...] = _swish(_dot(x_kj, wdown_ref[...]))


def _edge_pre(x1, rbfp, wji, bji, wkj, bkj, m, wdown):
    return pl.pallas_call(
        _edge_pre_body,
        grid=(E // EB,),
        in_specs=[_rows(EB, HC), _rows(EB, 8), _full((HC, HC)),
                  _full((1, HC)), _full((HC, HC)), _full((1, HC)),
                  _full((8, HC)), _full((HC, IE))],
        out_specs=[_rows(EB, HC), _rows(EB, IE)],
        out_shape=[jax.ShapeDtypeStruct((E, HC), jnp.float32),
                   jax.ShapeDtypeStruct((E, IE), jnp.float32)],
    )(x1, rbfp, wji, bji, wkj, bkj, m, wdown)


def _edge_post_body(agg_ref, xji_ref, x1_ref, rbfp_ref, wup_ref, *rest):
    (wb1, bb1, wb2, bb2, wl, bl, wa1, ba1, wa2, ba2, wa3, ba3, wa4, ba4,
     wrbf, e1_ref, e2_ref) = rest
    xkj = _swish(_dot(agg_ref[...], wup_ref[...]))
    e1 = xji_ref[...] + xkj
    e1 = e1 + _swish(_dot(_swish(_dot(e1, wb1[...]) + bb1[...]), wb2[...])
                     + bb2[...])
    e1 = _swish(_dot(e1, wl[...]) + bl[...]) + x1_ref[...]
    e1 = e1 + _swish(_dot(_swish(_dot(e1, wa1[...]) + ba1[...]), wa2[...])
                     + ba2[...])
    e1 = e1 + _swish(_dot(_swish(_dot(e1, wa3[...]) + ba3[...]), wa4[...])
                     + ba4[...])
    e1_ref[...] = e1
    e2_ref[...] = _dot(rbfp_ref[...], wrbf[...]) * e1


def _edge_post(agg, xji, x1, rbfp, wup, wlist):
    specs = ([_rows(EB, IE), _rows(EB, HC), _rows(EB, HC), _rows(EB, 8),
              _full((IE, HC))]
             + [_full((HC, HC)), _full((1, HC))] * 7
             + [_full((8, HC))])
    return pl.pallas_call(
        _edge_post_body,
        grid=(E // EB,),
        in_specs=specs,
        out_specs=[_rows(EB, HC)] * 2,
        out_shape=[jax.ShapeDtypeStruct((E, HC), jnp.float32)] * 2,
    )(agg, xji, x1, rbfp, wup, *wlist)


def _sproj_body(sbfp_ref, m_ref, out_ref):
    out_ref[...] = _dot(sbfp_ref[...], m_ref[...])


def _sbf_proj(sbfp, m):
    return pl.pallas_call(
        _sproj_body,
        grid=(T // TB,),
        in_specs=[_rows(TB, 48), _full((48, IE))],
        out_specs=_rows(TB, IE),
        out_shape=jax.ShapeDtypeStruct((T, IE), jnp.float32),
    )(sbfp, m)


def _tmul_body(g_ref, s_ref, out_ref):
    out_ref[...] = g_ref[...] * s_ref[...]


def _tmul(g, s):
    return pl.pallas_call(
        _tmul_body,
        grid=(T // TB,),
        in_specs=[_rows(TB, IE), _rows(TB, IE)],
        out_specs=_rows(TB, IE),
        out_shape=jax.ShapeDtypeStruct((T, IE), jnp.float32),
    )(g, s)


_NW = 32


def _sc_gather64(table, idx, nrows, chunk=1000):
    per_w = nrows // _NW
    nchunks = per_w // chunk
    assert per_w % chunk == 0 and chunk % 8 == 0
    mesh = plsc.VectorSubcoreMesh(core_axis_name="c", subcore_axis_name="s")

    @functools.partial(
        pl.kernel,
        out_type=jax.ShapeDtypeStruct((nrows, 64), jnp.float32),
        mesh=mesh,
        compiler_params=pltpu.CompilerParams(use_tc_tiling_on_sc=False),
        scratch_types=[
            pltpu.VMEM((chunk,), jnp.int32),
            pltpu.VMEM((chunk, 64), jnp.float32),
            pltpu.SemaphoreType.DMA,
        ],
    )
    def k(table_hbm, idx_hbm, out_hbm, idx_v, g_v, sem):
        wid = lax.axis_index("s") * 2 + lax.axis_index("c")
        base = wid * per_w
        for c in range(nchunks):
            off = base + c * chunk
            pltpu.sync_copy(idx_hbm.at[pl.ds(off, chunk)], idx_v)
            pltpu.async_copy(table_hbm.at[idx_v], g_v, sem).wait()
            pltpu.sync_copy(g_v, out_hbm.at[pl.ds(off, chunk)])

    return k(table, idx)


def _vmlp_body(vagg_ref, batch_ref, wup_ref, bup_ref, w1, b1, w2, b2, w3,
               b3, wo, u_ref):
    v = _dot(vagg_ref[...], wup_ref[...]) + bup_ref[...]
    v = _swish(_dot(v, w1[...]) + b1[...])
    v = _swish(_dot(v, w2[...]) + b2[...])
    v = _swish(_dot(v, w3[...]) + b3[...])
    v = _dot(v, wo[...])
    seg = batch_ref[...]
    onehot = (seg == jax.lax.broadcasted_iota(jnp.int32, (NB, NG), 1)
              ).astype(jnp.float32)
    part = jax.lax.dot_general(onehot, v, (((0,), (0,)), ((), ())),
                               preferred_element_type=jnp.float32)

    @pl.when(pl.program_id(0) == 0)
    def _():
        u_ref[...] = jnp.zeros_like(u_ref)

    u_ref[...] += part


def _vmlp(vagg, batch2d, V):
    wlist = [V['lin_up']['w'], V['lin_up']['b'].reshape(1, OE)]
    for l in V['lins']:
        wlist += [l['w'], l['b'].reshape(1, OE)]
    wlist += [V['lin']['w']]
    specs = ([_rows(NB, HC), _rows(NB, 1), _full((HC, OE)), _full((1, OE))]
             + [_full((OE, OE)), _full((1, OE))] * 3
             + [_full((OE, 1))])
    return pl.pallas_call(
        _vmlp_body,
        grid=(N // NB,),
        in_specs=specs,
        out_specs=_full((NG, 1)),
        out_shape=jax.ShapeDtypeStruct((NG, 1), jnp.float32),
    )(vagg, batch2d, *wlist)


def kernel(z, rbf, sbf, i, j, idx_kj, idx_ji, batch, params):
    P = params
    rbfp = jnp.pad(rbf, ((0, 0), (0, 2)))
    sbfp = jnp.pad(sbf, ((0, 0), (0, 6)))
    batch2d = batch.astype(jnp.int32).reshape(N, 1)

    x = P['emb_table'][z]

    Wfull = P['init']['lin']['w']
    wa, wb, wc = Wfull[:HC], Wfull[HC:2 * HC], Wfull[2 * HC:]
    xa, xb = _node_proj(x, wa, wb)
    gxa = jnp.take(xa, i, axis=0)
    gxb = jnp.take(xb, j, axis=0)
    w0 = jnp.pad(P['init']['lin_rbf_0']['w'], ((0, 2), (0, 0)))
    b0 = P['init']['lin_rbf_0']['b'].reshape(1, HC)
    bc = P['init']['lin']['b'].reshape(1, HC)
    wr1 = jnp.pad(P['init']['lin_rbf_1']['w'], ((0, 2), (0, 0)))
    e1, e2, rbf0 = _init_edge(gxa, gxb, rbfp, w0, b0, Wfull[2 * HC:], bc,
                              wr1)

    V0 = P['update_v'][0]
    vagg = jax.ops.segment_sum(e2, i, num_segments=N)
    u = _vmlp(vagg, batch2d, V0)

    for li in range(4):
        L = P['update_e'][li]
        m_rbf = jnp.pad(_dot(L['lin_rbf1']['w'], L['lin_rbf2']['w']),
                        ((0, 2), (0, 0)))
        m_sbf = jnp.pad(_dot(L['lin_sbf1']['w'], L['lin_sbf2']['w']),
                        ((0, 6), (0, 0)))
        w_rbf = jnp.pad(L['lin_rbf']['w'], ((0, 2), (0, 0)))

        xji, xdown = _edge_pre(
            e1, rbfp, L['lin_ji']['w'], L['lin_ji']['b'].reshape(1, HC),
            L['lin_kj']['w'], L['lin_kj']['b'].reshape(1, HC), m_rbf,
            L['lin_down']['w'])

        s = _sbf_proj(sbfp, m_sbf)
        g = _sc_gather64(xdown, idx_kj, T)
        prod = _tmul(g, s)
        agg = jax.ops.segment_sum(prod, idx_ji, num_segments=E)

        wlist = []
        for rl in L['before']:
            wlist += [rl['lin1']['w'], rl['lin1']['b'].reshape(1, HC),
                      rl['lin2']['w'], rl['lin2']['b'].reshape(1, HC)]
        wlist += [L['lin']['w'], L['lin']['b'].reshape(1, HC)]
        for rl in L['after']:
            wlist += [rl['lin1']['w'], rl['lin1']['b'].reshape(1, HC),
                      rl['lin2']['w'], rl['lin2']['b'].reshape(1, HC)]
        wlist += [w_rbf]
        e1, e2 = _edge_post(agg, xji, e1, rbfp, L['lin_up']['w'], wlist)

        vagg = jax.ops.segment_sum(e2, i, num_segments=N)
        u = u + _vmlp(vagg, batch2d, P['update_v'][li + 1])

    return u


# speedup vs baseline: 4.6132x; 1.1207x over previous
import functools

import jax
import jax.numpy as jnp
from jax import lax
from jax.experimental import pallas as pl
from jax.experimental.pallas import tpu as pltpu
from jax.experimental.pallas import tpu_sc as plsc

N = 10000
E = 160000
T = 640000
NG = 64
HC = 128
IE = 64
OE = 256

EB = 2000
TB = 4000
NB = 2000


def _swish(x):
    return x * jax.nn.sigmoid(x)


def _dot(a, b):
    return jnp.dot(a, b, preferred_element_type=jnp.float32)


def _full(shape):
    return pl.BlockSpec(shape, lambda e: tuple(0 for _ in shape))


def _rows(nrows, ncols):
    return pl.BlockSpec((nrows, ncols), lambda e: (e, 0))


def _node_body(x_ref, wa_ref, wb_ref, xa_ref, xb_ref):
    x = x_ref[...]
    xa_ref[...] = _dot(x, wa_ref[...])
    xb_ref[...] = _dot(x, wb_ref[...])


def _node_proj(x, wa, wb):
    return pl.pallas_call(
        _node_body,
        grid=(N // NB,),
        in_specs=[_rows(NB, HC), _full((HC, HC)), _full((HC, HC))],
        out_specs=[_rows(NB, HC), _rows(NB, HC)],
        out_shape=[jax.ShapeDtypeStruct((N, HC), jnp.float32)] * 2,
    )(x, wa, wb)


def _init_edge_body(gxa_ref, gxb_ref, rbfp_ref, w0_ref, b0_ref, wc_ref,
                    bc_ref, wr1_ref, e1_ref, e2_ref, rbf0_ref):
    rbfp = rbfp_ref[...]
    rbf0 = _swish(_dot(rbfp, w0_ref[...]) + b0_ref[...])
    e1 = _swish(gxa_ref[...] + gxb_ref[...] + _dot(rbf0, wc_ref[...])
                + bc_ref[...])
    e1_ref[...] = e1
    e2_ref[...] = _dot(rbfp, wr1_ref[...]) * e1
    rbf0_ref[...] = rbf0


def _init_edge(gxa, gxb, rbfp, w0, b0, wc, bc, wr1):
    return pl.pallas_call(
        _init_edge_body,
        grid=(E // EB,),
        in_specs=[_rows(EB, HC), _rows(EB, HC), _rows(EB, 8),
                  _full((8, HC)), _full((1, HC)), _full((HC, HC)),
                  _full((1, HC)), _full((8, HC))],
        out_specs=[_rows(EB, HC)] * 3,
        out_shape=[jax.ShapeDtypeStruct((E, HC), jnp.float32)] * 3,
    )(gxa, gxb, rbfp, w0, b0, wc, bc, wr1)


def _edge_pre_body(x1_ref, rbfp_ref, wji_ref, bji_ref, wkj_ref, bkj_ref,
                   m_ref, wdown_ref, xji_ref, xdown_ref):
    x1 = x1_ref[...]
    xji_ref[...] = _swish(_dot(x1, wji_ref[...]) + bji_ref[...])
    x_kj = _swish(_dot(x1, wkj_ref[...]) + bkj_ref[...])
    x_kj = x_kj * _dot(rbfp_ref[...], m_ref[...])
    xdown_ref[...] = _swish(_dot(x_kj, wdown_ref[...]))


def _edge_pre(x1, rbfp, wji, bji, wkj, bkj, m, wdown):
    return pl.pallas_call(
        _edge_pre_body,
        grid=(E // EB,),
        in_specs=[_rows(EB, HC), _rows(EB, 8), _full((HC, HC)),
                  _full((1, HC)), _full((HC, HC)), _full((1, HC)),
                  _full((8, HC)), _full((HC, IE))],
        out_specs=[_rows(EB, HC), _rows(EB, IE)],
        out_shape=[jax.ShapeDtypeStruct((E, HC), jnp.float32),
                   jax.ShapeDtypeStruct((E, IE), jnp.float32)],
    )(x1, rbfp, wji, bji, wkj, bkj, m, wdown)


def _edge_post_body(agg_ref, xji_ref, x1_ref, rbfp_ref, wup_ref, *rest):
    (wb1, bb1, wb2, bb2, wl, bl, wa1, ba1, wa2, ba2, wa3, ba3, wa4, ba4,
     wrbf, e1_ref, e2_ref) = rest
    xkj = _swish(_dot(agg_ref[...], wup_ref[...]))
    e1 = xji_ref[...] + xkj
    e1 = e1 + _swish(_dot(_swish(_dot(e1, wb1[...]) + bb1[...]), wb2[...])
                     + bb2[...])
    e1 = _swish(_dot(e1, wl[...]) + bl[...]) + x1_ref[...]
    e1 = e1 + _swish(_dot(_swish(_dot(e1, wa1[...]) + ba1[...]), wa2[...])
                     + ba2[...])
    e1 = e1 + _swish(_dot(_swish(_dot(e1, wa3[...]) + ba3[...]), wa4[...])
                     + ba4[...])
    e1_ref[...] = e1
    e2_ref[...] = _dot(rbfp_ref[...], wrbf[...]) * e1


def _edge_post(agg, xji, x1, rbfp, wup, wlist):
    specs = ([_rows(EB, IE), _rows(EB, HC), _rows(EB, HC), _rows(EB, 8),
              _full((IE, HC))]
             + [_full((HC, HC)), _full((1, HC))] * 7
             + [_full((8, HC))])
    return pl.pallas_call(
        _edge_post_body,
        grid=(E // EB,),
        in_specs=specs,
        out_specs=[_rows(EB, HC)] * 2,
        out_shape=[jax.ShapeDtypeStruct((E, HC), jnp.float32)] * 2,
    )(agg, xji, x1, rbfp, wup, *wlist)


def _sproj_body(sbfp_ref, m_ref, out_ref):
    out_ref[...] = _dot(sbfp_ref[...], m_ref[...])


def _sbf_proj(sbfp, m):
    return pl.pallas_call(
        _sproj_body,
        grid=(T // TB,),
        in_specs=[_rows(TB, 48), _full((48, IE))],
        out_specs=_rows(TB, IE),
        out_shape=jax.ShapeDtypeStruct((T, IE), jnp.float32),
    )(sbfp, m)


def _tmul_body(g_ref, s_ref, out_ref):
    out_ref[...] = g_ref[...] * s_ref[...]


def _tmul(g, s):
    return pl.pallas_call(
        _tmul_body,
        grid=(T // TB,),
        in_specs=[_rows(TB, IE), _rows(TB, IE)],
        out_specs=_rows(TB, IE),
        out_shape=jax.ShapeDtypeStruct((T, IE), jnp.float32),
    )(g, s)


_NW = 32


def _sc_gather64(table, idx, nrows, chunk=1000):
    per_w = nrows // _NW
    nchunks = per_w // chunk
    assert per_w % chunk == 0 and chunk % 8 == 0
    mesh = plsc.VectorSubcoreMesh(core_axis_name="c", subcore_axis_name="s")

    @functools.partial(
        pl.kernel,
        out_type=jax.ShapeDtypeStruct((nrows, 64), jnp.float32),
        mesh=mesh,
        compiler_params=pltpu.CompilerParams(use_tc_tiling_on_sc=False),
        scratch_types=[
            pltpu.VMEM((chunk,), jnp.int32),
            pltpu.VMEM((chunk, 64), jnp.float32),
            pltpu.SemaphoreType.DMA,
        ],
    )
    def k(table_hbm, idx_hbm, out_hbm, idx_v, g_v, sem):
        wid = lax.axis_index("s") * 2 + lax.axis_index("c")
        base = wid * per_w
        for c in range(nchunks):
            off = base + c * chunk
            pltpu.sync_copy(idx_hbm.at[pl.ds(off, chunk)], idx_v)
            pltpu.async_copy(table_hbm.at[idx_v], g_v, sem).wait()
            pltpu.sync_copy(g_v, out_hbm.at[pl.ds(off, chunk)])

    return k(table, idx)


def _sc_segsum_n(e2, idx, zeros_n):
    per_w = E // 16
    chunk = 1000
    mesh = plsc.VectorSubcoreMesh(core_axis_name="c", subcore_axis_name="s")

    @functools.partial(
        pl.kernel,
        out_type=jax.ShapeDtypeStruct((N, HC), jnp.float32),
        mesh=mesh,
        compiler_params=pltpu.CompilerParams(use_tc_tiling_on_sc=False),
        scratch_types=[
            pltpu.VMEM((chunk,), jnp.int32),
            pltpu.VMEM((chunk, IE), jnp.float32),
            pltpu.VMEM_SHARED((N, IE), jnp.float32),
            pltpu.SemaphoreType.DMA,
        ],
    )
    def k(e2_hbm, idx_hbm, zeros_hbm, out_hbm, idx_v, rows_v, acc_sh, sem):
        cid = lax.axis_index("c")
        sid = lax.axis_index("s")
        base = sid * per_w
        col = cid * IE

        @pl.when(sid == 0)
        def _():
            pltpu.sync_copy(zeros_hbm, acc_sh)

        plsc.subcore_barrier()
        for c in range(per_w // chunk):
            off = base + c * chunk
            pltpu.sync_copy(idx_hbm.at[pl.ds(off, chunk)], idx_v)
            pltpu.sync_copy(e2_hbm.at[pl.ds(off, chunk), pl.ds(col, IE)],
                            rows_v)
            pltpu.sync_copy(rows_v, acc_sh.at[idx_v], add=True)
        plsc.subcore_barrier()
        nslice = N // 16
        pltpu.sync_copy(acc_sh.at[pl.ds(sid * nslice, nslice)],
                        out_hbm.at[pl.ds(sid * nslice, nslice),
                                   pl.ds(col, IE)])

    return k(e2, idx, zeros_n)


def _vmlp_body(vagg_ref, batch_ref, wup_ref, bup_ref, w1, b1, w2, b2,
               w3, b3, wo, u_ref):
    v = _dot(vagg_ref[...], wup_ref[...]) + bup_ref[...]
    v = _swish(_dot(v, w1[...]) + b1[...])
    v = _swish(_dot(v, w2[...]) + b2[...])
    v = _swish(_dot(v, w3[...]) + b3[...])
    v = _dot(v, wo[...])
    seg = batch_ref[...]
    onehot = (seg == jax.lax.broadcasted_iota(jnp.int32, (NB, NG), 1)
              ).astype(jnp.float32)
    part = jax.lax.dot_general(onehot, v, (((0,), (0,)), ((), ())),
                               preferred_element_type=jnp.float32)

    @pl.when(pl.program_id(0) == 0)
    def _():
        u_ref[...] = jnp.zeros_like(u_ref)

    u_ref[...] += part


def _vmlp(vagg, batch2d, V):
    wlist = [V['lin_up']['w'], V['lin_up']['b'].reshape(1, OE)]
    for l in V['lins']:
        wlist += [l['w'], l['b'].reshape(1, OE)]
    wlist += [V['lin']['w']]
    specs = ([_rows(NB, HC), _rows(NB, 1), _full((HC, OE)), _full((1, OE))]
             + [_full((OE, OE)), _full((1, OE))] * 3
             + [_full((OE, 1))])
    return pl.pallas_call(
        _vmlp_body,
        grid=(N // NB,),
        in_specs=specs,
        out_specs=_full((NG, 1)),
        out_shape=jax.ShapeDtypeStruct((NG, 1), jnp.float32),
    )(vagg, batch2d, *wlist)


def kernel(z, rbf, sbf, i, j, idx_kj, idx_ji, batch, params):
    P = params
    rbfp = jnp.pad(rbf, ((0, 0), (0, 2)))
    sbfp = jnp.pad(sbf, ((0, 0), (0, 6)))
    batch2d = batch.astype(jnp.int32).reshape(N, 1)

    x = P['emb_table'][z]

    Wfull = P['init']['lin']['w']
    wa, wb, wc = Wfull[:HC], Wfull[HC:2 * HC], Wfull[2 * HC:]
    xa, xb = _node_proj(x, wa, wb)
    gxa = jnp.take(xa, i, axis=0)
    gxb = jnp.take(xb, j, axis=0)
    w0 = jnp.pad(P['init']['lin_rbf_0']['w'], ((0, 2), (0, 0)))
    b0 = P['init']['lin_rbf_0']['b'].reshape(1, HC)
    bc = P['init']['lin']['b'].reshape(1, HC)
    wr1 = jnp.pad(P['init']['lin_rbf_1']['w'], ((0, 2), (0, 0)))
    e1, e2, rbf0 = _init_edge(gxa, gxb, rbfp, w0, b0, Wfull[2 * HC:], bc,
                              wr1)

    zeros_n = jnp.zeros((N, IE), jnp.float32)
    i32 = i.astype(jnp.int32)
    vagg = _sc_segsum_n(e2, i32, zeros_n)
    u = _vmlp(vagg, batch2d, P['update_v'][0])

    for li in range(4):
        L = P['update_e'][li]
        m_rbf = jnp.pad(_dot(L['lin_rbf1']['w'], L['lin_rbf2']['w']),
                        ((0, 2), (0, 0)))
        m_sbf = jnp.pad(_dot(L['lin_sbf1']['w'], L['lin_sbf2']['w']),
                        ((0, 6), (0, 0)))
        w_rbf = jnp.pad(L['lin_rbf']['w'], ((0, 2), (0, 0)))

        xji, xdown = _edge_pre(
            e1, rbfp, L['lin_ji']['w'], L['lin_ji']['b'].reshape(1, HC),
            L['lin_kj']['w'], L['lin_kj']['b'].reshape(1, HC), m_rbf,
            L['lin_down']['w'])

        s = _sbf_proj(sbfp, m_sbf)
        g = _sc_gather64(xdown, idx_kj, T)
        prod = _tmul(g, s)
        agg = jax.ops.segment_sum(prod, idx_ji, num_segments=E)

        wlist = []
        for rl in L['before']:
            wlist += [rl['lin1']['w'], rl['lin1']['b'].reshape(1, HC),
                      rl['lin2']['w'], rl['lin2']['b'].reshape(1, HC)]
        wlist += [L['lin']['w'], L['lin']['b'].reshape(1, HC)]
        for rl in L['after']:
            wlist += [rl['lin1']['w'], rl['lin1']['b'].reshape(1, HC),
                      rl['lin2']['w'], rl['lin2']['b'].reshape(1, HC)]
        wlist += [w_rbf]
        e1, e2 = _edge_post(agg, xji, e1, rbfp, L['lin_up']['w'], wlist)

        vagg = _sc_segsum_n(e2, i32, zeros_n)
        u = u + _vmlp(vagg, batch2d, P['update_v'][li + 1])

    return u


# speedup vs baseline: 4.7318x; 1.0257x over previous
import functools

import jax
import jax.numpy as jnp
from jax import lax
from jax.experimental import pallas as pl
from jax.experimental.pallas import tpu as pltpu
from jax.experimental.pallas import tpu_sc as plsc

N = 10000
E = 160000
T = 640000
NG = 64
HC = 128
IE = 64
OE = 256

EB = 2000
TB = 4000
NB = 2000


def _swish(x):
    return x * jax.nn.sigmoid(x)


def _dot(a, b):
    return jnp.dot(a, b, preferred_element_type=jnp.float32)


def _full(shape):
    return pl.BlockSpec(shape, lambda e: tuple(0 for _ in shape))


def _rows(nrows, ncols):
    return pl.BlockSpec((nrows, ncols), lambda e: (e, 0))


def _node_body(x_ref, wa_ref, wb_ref, xa_ref, xb_ref):
    x = x_ref[...]
    xa_ref[...] = _dot(x, wa_ref[...])
    xb_ref[...] = _dot(x, wb_ref[...])


def _node_proj(x, wa, wb):
    return pl.pallas_call(
        _node_body,
        grid=(N // NB,),
        in_specs=[_rows(NB, HC), _full((HC, HC)), _full((HC, HC))],
        out_specs=[_rows(NB, HC), _rows(NB, HC)],
        out_shape=[jax.ShapeDtypeStruct((N, HC), jnp.float32)] * 2,
    )(x, wa, wb)


def _init_edge_body(gxa_ref, gxb_ref, rbfp_ref, w0_ref, b0_ref, wc_ref,
                    bc_ref, wr1_ref, e1_ref, e2_ref, rbf0_ref):
    rbfp = rbfp_ref[...]
    rbf0 = _swish(_dot(rbfp, w0_ref[...]) + b0_ref[...])
    e1 = _swish(gxa_ref[...] + gxb_ref[...] + _dot(rbf0, wc_ref[...])
                + bc_ref[...])
    e1_ref[...] = e1
    e2_ref[...] = _dot(rbfp, wr1_ref[...]) * e1
    rbf0_ref[...] = rbf0


def _init_edge(gxa, gxb, rbfp, w0, b0, wc, bc, wr1):
    return pl.pallas_call(
        _init_edge_body,
        grid=(E // EB,),
        in_specs=[_rows(EB, HC), _rows(EB, HC), _rows(EB, 8),
                  _full((8, HC)), _full((1, HC)), _full((HC, HC)),
                  _full((1, HC)), _full((8, HC))],
        out_specs=[_rows(EB, HC)] * 3,
        out_shape=[jax.ShapeDtypeStruct((E, HC), jnp.float32)] * 3,
    )(gxa, gxb, rbfp, w0, b0, wc, bc, wr1)


def _edge_pre_body(x1_ref, rbfp_ref, wji_ref, bji_ref, wkj_ref, bkj_ref,
                   m_ref, wdown_ref, xji_ref, xdown_ref):
    x1 = x1_ref[...]
    xji_ref[...] = _swish(_dot(x1, wji_ref[...]) + bji_ref[...])
    x_kj = _swish(_dot(x1, wkj_ref[...]) + bkj_ref[...])
    x_kj = x_kj * _dot(rbfp_ref[...], m_ref[...])
    xdown_ref[...] = _swish(_dot(x_kj, wdown_ref[...]))


def _edge_pre(x1, rbfp, wji, bji, wkj, bkj, m, wdown):
    return pl.pallas_call(
        _edge_pre_body,
        grid=(E // EB,),
        in_specs=[_rows(EB, HC), _rows(EB, 8), _full((HC, HC)),
                  _full((1, HC)), _full((HC, HC)), _full((1, HC)),
                  _full((8, HC)), _full((HC, IE))],
        out_specs=[_rows(EB, HC), _rows(EB, IE)],
        out_shape=[jax.ShapeDtypeStruct((E, HC), jnp.float32),
                   jax.ShapeDtypeStruct((E, IE), jnp.float32)],
    )(x1, rbfp, wji, bji, wkj, bkj, m, wdown)


def _edge_post_body(agg_ref, xji_ref, x1_ref, rbfp_ref, wup_ref, *rest):
    (wb1, bb1, wb2, bb2, wl, bl, wa1, ba1, wa2, ba2, wa3, ba3, wa4, ba4,
     wrbf, e1_ref, e2_ref) = rest
    xkj = _swish(_dot(agg_ref[...], wup_ref[...]))
    e1 = xji_ref[...] + xkj
    e1 = e1 + _swish(_dot(_swish(_dot(e1, wb1[...]) + bb1[...]), wb2[...])
                     + bb2[...])
    e1 = _swish(_dot(e1, wl[...]) + bl[...]) + x1_ref[...]
    e1 = e1 + _swish(_dot(_swish(_dot(e1, wa1[...]) + ba1[...]), wa2[...])
                     + ba2[...])
    e1 = e1 + _swish(_dot(_swish(_dot(e1, wa3[...]) + ba3[...]), wa4[...])
                     + ba4[...])
    e1_ref[...] = e1
    e2_ref[...] = _dot(rbfp_ref[...], wrbf[...]) * e1


def _edge_post(agg, xji, x1, rbfp, wup, wlist):
    specs = ([_rows(EB, IE), _rows(EB, HC), _rows(EB, HC), _rows(EB, 8),
              _full((IE, HC))]
             + [_full((HC, HC)), _full((1, HC))] * 7
             + [_full((8, HC))])
    return pl.pallas_call(
        _edge_post_body,
        grid=(E // EB,),
        in_specs=specs,
        out_specs=[_rows(EB, HC)] * 2,
        out_shape=[jax.ShapeDtypeStruct((E, HC), jnp.float32)] * 2,
    )(agg, xji, x1, rbfp, wup, *wlist)


def _sproj_body(sbfp_ref, m_ref, out_ref):
    out_ref[...] = _dot(sbfp_ref[...], m_ref[...])


def _sbf_proj(sbfp, m):
    return pl.pallas_call(
        _sproj_body,
        grid=(T // TB,),
        in_specs=[_rows(TB, 48), _full((48, IE))],
        out_specs=_rows(TB, IE),
        out_shape=jax.ShapeDtypeStruct((T, IE), jnp.float32),
    )(sbfp, m)


def _tmul_body(g_ref, s_ref, out_ref):
    out_ref[...] = g_ref[...] * s_ref[...]


def _tmul(g, s):
    return pl.pallas_call(
        _tmul_body,
        grid=(T // TB,),
        in_specs=[_rows(TB, IE), _rows(TB, IE)],
        out_specs=_rows(TB, IE),
        out_shape=jax.ShapeDtypeStruct((T, IE), jnp.float32),
    )(g, s)


_NW = 32


def _sc_gather64(table, idx, nrows, chunk=1000):
    per_w = nrows // _NW
    nchunks = per_w // chunk
    assert per_w % chunk == 0 and chunk % 8 == 0
    mesh = plsc.VectorSubcoreMesh(core_axis_name="c", subcore_axis_name="s")

    @functools.partial(
        pl.kernel,
        out_type=jax.ShapeDtypeStruct((nrows, 64), jnp.float32),
        mesh=mesh,
        compiler_params=pltpu.CompilerParams(use_tc_tiling_on_sc=False),
        scratch_types=[
            pltpu.VMEM((chunk,), jnp.int32),
            pltpu.VMEM((chunk, 64), jnp.float32),
            pltpu.SemaphoreType.DMA,
        ],
    )
    def k(table_hbm, idx_hbm, out_hbm, idx_v, g_v, sem):
        wid = lax.axis_index("s") * 2 + lax.axis_index("c")
        base = wid * per_w
        for c in range(nchunks):
            off = base + c * chunk
            pltpu.sync_copy(idx_hbm.at[pl.ds(off, chunk)], idx_v)
            pltpu.async_copy(table_hbm.at[idx_v], g_v, sem).wait()
            pltpu.sync_copy(g_v, out_hbm.at[pl.ds(off, chunk)])

    return k(table, idx)


_GDN = lax.GatherDimensionNumbers(offset_dims=(), collapsed_slice_dims=(0,),
                                  start_index_map=(0,))


def _psum16(x):
    iv = jnp.arange(16, dtype=jnp.int32)
    c = x
    for kk in (1, 2, 4, 8):
        ind = 1 - lax.shift_right_logical(iv - kk, 31)
        idxv = (iv - kk) * ind
        g = lax.gather(c, idxv[:, None], _GDN, (1,),
                       mode=lax.GatherScatterMode.PROMISE_IN_BOUNDS)
        c = c + g * ind
    return c


RNG = 14080
NPASS = 12
EPAD = RNG * NPASS
ACCR = RNG + 8
DUMMY = RNG + 2
TSEG = 8000
CAP = TSEG + 160
CH = 128


def _sc_triplet_agg(xdown, s, idx_kj, idx_ji):
    per_tile = T // 16
    nseg = per_tile // TSEG
    mesh = plsc.VectorSubcoreMesh(core_axis_name="c", subcore_axis_name="s")

    @functools.partial(
        pl.kernel,
        out_type=jax.ShapeDtypeStruct((EPAD, IE), jnp.float32),
        mesh=mesh,
        compiler_params=pltpu.CompilerParams(
            use_tc_tiling_on_sc=False, needs_layout_passes=False),
        scratch_types=[
            pltpu.VMEM((TSEG,), jnp.int32),
            pltpu.VMEM((TSEG,), jnp.int32),
            pltpu.VMEM((CAP,), jnp.int32),
            pltpu.VMEM((CAP,), jnp.int32),
            pltpu.VMEM((CAP,), jnp.int32),
            pltpu.VMEM((CH, IE), jnp.float32),
            pltpu.VMEM((CH, IE), jnp.float32),
            pltpu.VMEM((1, CH), jnp.int32),
            pltpu.VMEM((256, IE), jnp.float32),
            pltpu.VMEM_SHARED((ACCR, IE), jnp.float32),
            pltpu.SemaphoreType.DMA,
            pltpu.SemaphoreType.DMA,
        ],
    )
    def k(xd_hbm, s_hbm, kj_hbm, ji_hbm, out_hbm, ji_v, kj_v, tgt_v,
          tpos_v, kjc_v, g_buf, s_buf, idx2d, zero_buf, acc_sh, sem1,
          sem2):
        cid = lax.axis_index("c")
        sid = lax.axis_index("s")
        zvec = jnp.zeros((16,), jnp.float32)

        def _zb(r, _):
            for q in range(IE // 16):
                zero_buf[r, pl.ds(q * 16, 16)] = zvec
            return 0

        lax.fori_loop(0, 256, _zb, 0)
        wrows = RNG // 16

        for kk in range(6):
            lo = (2 * kk + cid) * RNG
            if True:
                hi = lo + RNG
                row0 = sid * wrows
                for zz in range(3):
                    pltpu.sync_copy(zero_buf,
                                    acc_sh.at[pl.ds(row0 + zz * 256, 256)])
                pltpu.sync_copy(zero_buf.at[pl.ds(0, 112)],
                                acc_sh.at[pl.ds(row0 + 768, 112)])
                plsc.subcore_barrier()

                for seg in range(nseg):
                    seg_base = sid * per_tile + seg * TSEG
                    pltpu.sync_copy(ji_hbm.at[pl.ds(seg_base, TSEG)], ji_v)
                    pltpu.sync_copy(kj_hbm.at[pl.ds(seg_base, TSEG)], kj_v)

                    def _scan(kq, pos):
                        off = kq * 16
                        jv = ji_v[pl.ds(off, 16)]
                        kv = kj_v[pl.ds(off, 16)]
                        sa = lax.shift_right_logical(jv - lo, 31)
                        sb = lax.shift_right_logical(jv - hi, 31)
                        mi = (1 - sa) * sb
                        cum = _psum16(mi)
                        dst = (mi * (pos + cum - 1)
                               + (1 - mi) * (CAP - 1))
                        tglob = (seg_base + off
                                 + jnp.arange(16, dtype=jnp.int32))
                        plsc.store_scatter(tgt_v, [dst], jv - lo)
                        plsc.store_scatter(tpos_v, [dst], tglob)
                        plsc.store_scatter(kjc_v, [dst], kv)
                        return pos + cum[15]

                    pos = lax.fori_loop(0, TSEG // 16, _scan, 0)

                    dvec = jnp.full((16,), DUMMY, jnp.int32)
                    zivec = jnp.zeros((16,), jnp.int32)
                    for kq in range(9):
                        offp = pos + kq * 16
                        tgt_v[pl.ds(offp, 16)] = dvec
                        tpos_v[pl.ds(offp, 16)] = zivec
                        kjc_v[pl.ds(offp, 16)] = zivec

                    nch = (pos + CH - 1) // CH

                    def _chunk(c, _):
                        cb = c * CH
                        cp1 = pltpu.async_copy(
                            xd_hbm.at[kjc_v.at[pl.ds(cb, CH)]], g_buf,
                            sem1)
                        cp2 = pltpu.async_copy(
                            s_hbm.at[tpos_v.at[pl.ds(cb, CH)]], s_buf,
                            sem2)
                        cp1.wait()
                        cp2.wait()

                        def _mul(mq, __):
                            o = mq * 16
                            r = mq // (IE // 16)
                            q = o - r * IE
                            g_buf[r, pl.ds(q, 16)] = (
                                g_buf[r, pl.ds(q, 16)]
                                * s_buf[r, pl.ds(q, 16)])
                            return 0

                        lax.fori_loop(0, CH * IE // 16, _mul, 0)

                        def _cpi(q, __):
                            idx2d[0, pl.ds(q * 16, 16)] = (
                                tgt_v[pl.ds(cb + q * 16, 16)])
                            return 0

                        lax.fori_loop(0, CH // 16, _cpi, 0)
                        pltpu.sync_copy(g_buf, acc_sh.at[idx2d.at[0]],
                                        add=True)
                        return 0

                    lax.fori_loop(0, nch, _chunk, 0)

                plsc.subcore_barrier()
                pltpu.sync_copy(
                    acc_sh.at[pl.ds(sid * wrows, wrows)],
                    out_hbm.at[pl.ds(lo + sid * wrows, wrows)])
                plsc.subcore_barrier()

    return k(xdown, s, idx_kj, idx_ji)


def _sc_segsum_n(e2, idx, zeros_n):
    per_w = E // 16
    chunk = 1000
    mesh = plsc.VectorSubcoreMesh(core_axis_name="c", subcore_axis_name="s")

    @functools.partial(
        pl.kernel,
        out_type=jax.ShapeDtypeStruct((N, HC), jnp.float32),
        mesh=mesh,
        compiler_params=pltpu.CompilerParams(use_tc_tiling_on_sc=False),
        scratch_types=[
            pltpu.VMEM((chunk,), jnp.int32),
            pltpu.VMEM((chunk, IE), jnp.float32),
            pltpu.VMEM_SHARED((N, IE), jnp.float32),
            pltpu.SemaphoreType.DMA,
        ],
    )
    def k(e2_hbm, idx_hbm, zeros_hbm, out_hbm, idx_v, rows_v, acc_sh, sem):
        cid = lax.axis_index("c")
        sid = lax.axis_index("s")
        base = sid * per_w
        col = cid * IE

        @pl.when(sid == 0)
        def _():
            pltpu.sync_copy(zeros_hbm, acc_sh)

        plsc.subcore_barrier()
        for c in range(per_w // chunk):
            off = base + c * chunk
            pltpu.sync_copy(idx_hbm.at[pl.ds(off, chunk)], idx_v)
            pltpu.sync_copy(e2_hbm.at[pl.ds(off, chunk), pl.ds(col, IE)],
                            rows_v)
            pltpu.sync_copy(rows_v, acc_sh.at[idx_v], add=True)
        plsc.subcore_barrier()
        nslice = N // 16
        pltpu.sync_copy(acc_sh.at[pl.ds(sid * nslice, nslice)],
                        out_hbm.at[pl.ds(sid * nslice, nslice),
                                   pl.ds(col, IE)])

    return k(e2, idx, zeros_n)


def _vmlp_body(vagg_ref, batch_ref, wup_ref, bup_ref, w1, b1, w2, b2,
               w3, b3, wo, u_ref):
    v = _dot(vagg_ref[...], wup_ref[...]) + bup_ref[...]
    v = _swish(_dot(v, w1[...]) + b1[...])
    v = _swish(_dot(v, w2[...]) + b2[...])
    v = _swish(_dot(v, w3[...]) + b3[...])
    v = _dot(v, wo[...])
    seg = batch_ref[...]
    onehot = (seg == jax.lax.broadcasted_iota(jnp.int32, (NB, NG), 1)
              ).astype(jnp.float32)
    part = jax.lax.dot_general(onehot, v, (((0,), (0,)), ((), ())),
                               preferred_element_type=jnp.float32)

    @pl.when(pl.program_id(0) == 0)
    def _():
        u_ref[...] = jnp.zeros_like(u_ref)

    u_ref[...] += part


def _vmlp(vagg, batch2d, V):
    wlist = [V['lin_up']['w'], V['lin_up']['b'].reshape(1, OE)]
    for l in V['lins']:
        wlist += [l['w'], l['b'].reshape(1, OE)]
    wlist += [V['lin']['w']]
    specs = ([_rows(NB, HC), _rows(NB, 1), _full((HC, OE)), _full((1, OE))]
             + [_full((OE, OE)), _full((1, OE))] * 3
             + [_full((OE, 1))])
    return pl.pallas_call(
        _vmlp_body,
        grid=(N // NB,),
        in_specs=specs,
        out_specs=_full((NG, 1)),
        out_shape=jax.ShapeDtypeStruct((NG, 1), jnp.float32),
    )(vagg, batch2d, *wlist)


def kernel(z, rbf, sbf, i, j, idx_kj, idx_ji, batch, params):
    P = params
    rbfp = jnp.pad(rbf, ((0, 0), (0, 2)))
    sbfp = jnp.pad(sbf, ((0, 0), (0, 6)))
    batch2d = batch.astype(jnp.int32).reshape(N, 1)

    x = P['emb_table'][z]

    Wfull = P['init']['lin']['w']
    wa, wb, wc = Wfull[:HC], Wfull[HC:2 * HC], Wfull[2 * HC:]
    xa, xb = _node_proj(x, wa, wb)
    gxa = jnp.take(xa, i, axis=0)
    gxb = jnp.take(xb, j, axis=0)
    w0 = jnp.pad(P['init']['lin_rbf_0']['w'], ((0, 2), (0, 0)))
    b0 = P['init']['lin_rbf_0']['b'].reshape(1, HC)
    bc = P['init']['lin']['b'].reshape(1, HC)
    wr1 = jnp.pad(P['init']['lin_rbf_1']['w'], ((0, 2), (0, 0)))
    e1, e2, rbf0 = _init_edge(gxa, gxb, rbfp, w0, b0, Wfull[2 * HC:], bc,
                              wr1)

    zeros_n = jnp.zeros((N, IE), jnp.float32)
    i32 = i.astype(jnp.int32)
    kj32 = idx_kj.astype(jnp.int32)
    ji32 = idx_ji.astype(jnp.int32)
    vagg = _sc_segsum_n(e2, i32, zeros_n)
    u = _vmlp(vagg, batch2d, P['update_v'][0])

    for li in range(4):
        L = P['update_e'][li]
        m_rbf = jnp.pad(_dot(L['lin_rbf1']['w'], L['lin_rbf2']['w']),
                        ((0, 2), (0, 0)))
        m_sbf = jnp.pad(_dot(L['lin_sbf1']['w'], L['lin_sbf2']['w']),
                        ((0, 6), (0, 0)))
        w_rbf = jnp.pad(L['lin_rbf']['w'], ((0, 2), (0, 0)))

        xji, xdown = _edge_pre(
            e1, rbfp, L['lin_ji']['w'], L['lin_ji']['b'].reshape(1, HC),
            L['lin_kj']['w'], L['lin_kj']['b'].reshape(1, HC), m_rbf,
            L['lin_down']['w'])

        s = _sbf_proj(sbfp, m_sbf)
        agg = _sc_triplet_agg(xdown, s, kj32, ji32)[:E]

        wlist = []
        for rl in L['before']:
            wlist += [rl['lin1']['w'], rl['lin1']['b'].reshape(1, HC),
                      rl['lin2']['w'], rl['lin2']['b'].reshape(1, HC)]
        wlist += [L['lin']['w'], L['lin']['b'].reshape(1, HC)]
        for rl in L['after']:
            wlist += [rl['lin1']['w'], rl['lin1']['b'].reshape(1, HC),
                      rl['lin2']['w'], rl['lin2']['b'].reshape(1, HC)]
        wlist += [w_rbf]
        e1, e2 = _edge_post(agg, xji, e1, rbfp, L['lin_up']['w'], wlist)

        vagg = _sc_segsum_n(e2, i32, zeros_n)
        u = u + _vmlp(vagg, batch2d, P['update_v'][li + 1])

    return u
